# baseline (speedup 1.0000x reference)
import functools

import jax
import jax.numpy as jnp
from jax import lax
from jax.experimental import pallas as pl
from jax.experimental.pallas import tpu as pltpu
from jax.experimental.pallas import tpu_sc as plsc

DIM = 128
VOCAB = 6
B = 4096
L = 200

NC = 2
NS = 16
NW = NC * NS
NSPLIT = 2
BSUB = B // NSPLIT
RPW = BSUB // NW
NCHUNK = 13
SEQPAD = 208
REP = 16


def _compute_row(seq_slot, out_slot, table_v):
    iota = lax.iota(jnp.int32, 16)
    zero = iota * 0
    for c in range(NCHUNK):
        sv = seq_slot[pl.ds(c * 16, 16)]
        lv = iota + (c * 16)
        idx0 = sv * (DIM * REP) + iota
        tail = c == NCHUNK - 1
        mask = iota < (L - 16 * (NCHUNK - 1)) if tail else None

        @plsc.parallel_loop(0, DIM, step=1, unroll=8, carry=(idx0, zero))
        def _dbody(_, carry):
            idxv, dv = carry
            if tail:
                val = plsc.load_gather(table_v, [idxv], mask=mask)
                plsc.store_scatter(out_slot, [dv, lv], val, mask=mask)
            else:
                val = plsc.load_gather(table_v, [idxv])
                plsc.store_scatter(out_slot, [dv, lv], val)
            return (idxv + REP, dv + 1)


def _sc_body(seq_hbm, table_hbm, out_hbm, table_v, seq_v0, seq_v1,
             out_v0, out_v1, sem_t, sem_s0, sem_s1, sem_o0, sem_o1):
    seq_v = (seq_v0, seq_v1)
    out_v = (out_v0, out_v1)
    sem_s = (sem_s0, sem_s1)
    sem_o = (sem_o0, sem_o1)
    wid = lax.axis_index("s") * NC + lax.axis_index("c")
    base = wid * RPW

    pltpu.async_copy(table_hbm, table_v, sem_t)
    for s in (0, 1):
        r = base + s
        pltpu.async_copy(
            seq_hbm.at[pl.ds(r * L, L)], seq_v[s].at[pl.ds(0, L)], sem_s[s]
        )
    pltpu.make_async_copy(table_hbm, table_v, sem_t).wait()

    def iter_body(i, _):
        for s in (0, 1):
            r = base + 2 * i + s
            pltpu.make_async_copy(
                seq_hbm.at[pl.ds(r * L, L)],
                seq_v[s].at[pl.ds(0, L)],
                sem_s[s],
            ).wait()

            @pl.when(i > 0)
            def _wait_prev():
                pltpu.make_async_copy(
                    out_v[s], out_hbm.at[r - 2], sem_o[s]
                ).wait()

            _compute_row(seq_v[s], out_v[s], table_v)
            pltpu.async_copy(out_v[s], out_hbm.at[r], sem_o[s])

            @pl.when(2 * i + s + 2 < RPW)
            def _prefetch():
                pltpu.async_copy(
                    seq_hbm.at[pl.ds((r + 2) * L, L)],
                    seq_v[s].at[pl.ds(0, L)],
                    sem_s[s],
                )
        return 0

    lax.fori_loop(0, RPW // 2, iter_body, 0)
    for s in (0, 1):
        r = base + RPW - 2 + s
        pltpu.make_async_copy(out_v[s], out_hbm.at[r], sem_o[s]).wait()


@functools.cache
def _sc_call():
    return pl.kernel(
        _sc_body,
        out_type=jax.ShapeDtypeStruct((BSUB, DIM, L), jnp.float32),
        mesh=plsc.VectorSubcoreMesh(
            core_axis_name="c", subcore_axis_name="s",
            num_cores=NC, num_subcores=NS,
        ),
        scratch_types=[
            pltpu.VMEM((VOCAB * DIM * REP,), jnp.float32),
            pltpu.VMEM((SEQPAD,), jnp.int32),
            pltpu.VMEM((SEQPAD,), jnp.int32),
            pltpu.VMEM((DIM, L), jnp.float32),
            pltpu.VMEM((DIM, L), jnp.float32),
            pltpu.SemaphoreType.DMA,
            pltpu.SemaphoreType.DMA,
            pltpu.SemaphoreType.DMA,
            pltpu.SemaphoreType.DMA,
            pltpu.SemaphoreType.DMA,
        ],
        compiler_params=pltpu.CompilerParams(
            needs_layout_passes=False, use_tc_tiling_on_sc=True
        ),
    )


def kernel(seq, table):
    seq_flat = seq.astype(jnp.int32).reshape(B * L)
    tab_rep = jnp.broadcast_to(
        table.astype(jnp.float32).reshape(VOCAB * DIM, 1), (VOCAB * DIM, REP)
    ).reshape(VOCAB * DIM * REP)
    call = _sc_call()
    parts = [
        call(seq_flat[h * BSUB * L : (h + 1) * BSUB * L], tab_rep)
        for h in range(NSPLIT)
    ]
    return jnp.concatenate(parts, axis=0)


# speedup vs baseline: 1.4283x; 1.4283x over previous
import functools

import jax
import jax.numpy as jnp
from jax import lax
from jax.experimental import pallas as pl
from jax.experimental.pallas import tpu as pltpu
from jax.experimental.pallas import tpu_sc as plsc

DIM = 128
VOCAB = 6
B = 4096
L = 200

NC = 2
NS = 16
NW = NC * NS
RPW = B // NW
NCHUNK = 13
SEQPAD = 208
REP = 16


def _compute_row(seq_slot, out_slot, table_v):
    iota = lax.iota(jnp.int32, 16)
    zero = iota * 0
    for c in range(NCHUNK):
        sv = seq_slot[pl.ds(c * 16, 16)]
        lv = iota + (c * 16)
        idx0 = sv * (DIM * REP) + iota
        tail = c == NCHUNK - 1
        mask = iota < (L - 16 * (NCHUNK - 1)) if tail else None

        @plsc.parallel_loop(0, DIM, step=1, unroll=8, carry=(idx0, zero))
        def _dbody(_, carry):
            idxv, dv = carry
            if tail:
                val = plsc.load_gather(table_v, [idxv], mask=mask)
                plsc.store_scatter(out_slot, [dv, lv], val, mask=mask)
            else:
                val = plsc.load_gather(table_v, [idxv])
                plsc.store_scatter(out_slot, [dv, lv], val)
            return (idxv + REP, dv + 1)


def _sc_body(seq_hbm, table_hbm, out_hbm, table_v, seq_v0, seq_v1,
             out_v0, out_v1, sem_t, sem_s0, sem_s1, sem_o0, sem_o1):
    seq_v = (seq_v0, seq_v1)
    out_v = (out_v0, out_v1)
    sem_s = (sem_s0, sem_s1)
    sem_o = (sem_o0, sem_o1)
    wid = lax.axis_index("s") * NC + lax.axis_index("c")
    base = wid * RPW

    pltpu.async_copy(table_hbm, table_v, sem_t)
    for s in (0, 1):
        r = base + s
        pltpu.async_copy(
            seq_hbm.at[pl.ds(r * L, L)], seq_v[s].at[pl.ds(0, L)], sem_s[s]
        )
    pltpu.make_async_copy(table_hbm, table_v, sem_t).wait()

    def iter_body(i, _):
        for s in (0, 1):
            r = base + 2 * i + s
            pltpu.make_async_copy(
                seq_hbm.at[pl.ds(r * L, L)],
                seq_v[s].at[pl.ds(0, L)],
                sem_s[s],
            ).wait()

            @pl.when(i > 0)
            def _wait_prev():
                pltpu.make_async_copy(
                    out_v[s], out_hbm.at[r - 2], sem_o[s]
                ).wait()

            _compute_row(seq_v[s], out_v[s], table_v)
            pltpu.async_copy(out_v[s], out_hbm.at[r], sem_o[s])

            @pl.when(2 * i + s + 2 < RPW)
            def _prefetch():
                pltpu.async_copy(
                    seq_hbm.at[pl.ds((r + 2) * L, L)],
                    seq_v[s].at[pl.ds(0, L)],
                    sem_s[s],
                )
        return 0

    lax.fori_loop(0, RPW // 2, iter_body, 0)
    for s in (0, 1):
        r = base + RPW - 2 + s
        pltpu.make_async_copy(out_v[s], out_hbm.at[r], sem_o[s]).wait()


@functools.cache
def _sc_call():
    return pl.kernel(
        _sc_body,
        out_type=jax.ShapeDtypeStruct((B, DIM, L), jnp.float32),
        mesh=plsc.VectorSubcoreMesh(
            core_axis_name="c", subcore_axis_name="s",
            num_cores=NC, num_subcores=NS,
        ),
        scratch_types=[
            pltpu.VMEM((VOCAB * DIM * REP,), jnp.float32),
            pltpu.VMEM((SEQPAD,), jnp.int32),
            pltpu.VMEM((SEQPAD,), jnp.int32),
            pltpu.VMEM((DIM, L), jnp.float32),
            pltpu.VMEM((DIM, L), jnp.float32),
            pltpu.SemaphoreType.DMA,
            pltpu.SemaphoreType.DMA,
            pltpu.SemaphoreType.DMA,
            pltpu.SemaphoreType.DMA,
            pltpu.SemaphoreType.DMA,
        ],
        compiler_params=pltpu.CompilerParams(needs_layout_passes=False),
    )


def kernel(seq, table):
    seq_flat = seq.astype(jnp.int32).reshape(B * L)
    tab_rep = jnp.broadcast_to(
        table.astype(jnp.float32).reshape(VOCAB * DIM, 1), (VOCAB * DIM, REP)
    ).reshape(VOCAB * DIM * REP)
    return _sc_call()(seq_flat, tab_rep)
